# baseline (speedup 1.0000x reference)
import functools

import jax
import jax.numpy as jnp
from jax import lax
from jax.experimental import pallas as pl
from jax.experimental.pallas import tpu as pltpu
from jax.experimental.pallas import tpu_sc as plsc

NC, NS = 2, 16
NW = NC * NS
D = 128

_MESH = functools.partial(
    plsc.VectorSubcoreMesh, core_axis_name="c", subcore_axis_name="s",
    num_cores=NC, num_subcores=NS)


def _emb_gather(tables, ids_flat, npad):
    rows_per_tile = npad // NS
    chunk = 80
    nchunks = rows_per_tile // chunk

    @functools.partial(
        pl.kernel,
        out_type=jax.ShapeDtypeStruct((2 * npad, D), jnp.float32),
        mesh=_MESH(),
        scratch_types=[pltpu.VMEM((chunk,), jnp.int32),
                       pltpu.VMEM((chunk, D), jnp.float32),
                       pltpu.SemaphoreType.DMA],
    )
    def k(tab, ids, out, idx_v, rows_v, sem):
        c = lax.axis_index("c")
        s = lax.axis_index("s")
        base = c * npad + s * rows_per_tile

        def body(i, carry):
            off = base + i * chunk
            pltpu.sync_copy(ids.at[pl.ds(off, chunk)], idx_v)
            pltpu.async_copy(tab.at[idx_v], rows_v, sem).wait()
            pltpu.sync_copy(rows_v, out.at[pl.ds(off, chunk)])
            return carry
        lax.fori_loop(0, nchunks, body, 0)

    return k(tables, ids_flat)


def _lin_tc(X, EMB, WL, BL):
    n = X.shape[1]
    bs = 2000
    grid = (2, n // bs)

    def body(x_ref, e_ref, w_ref, b_ref, o_ref):
        o_ref[...] = (jnp.dot(x_ref[0], w_ref[0],
                              preferred_element_type=jnp.float32)
                      + b_ref[0] + e_ref[0])[None]

    return pl.pallas_call(
        body,
        grid=grid,
        in_specs=[
            pl.BlockSpec((1, bs, D), lambda e, i: (e, i, 0)),
            pl.BlockSpec((1, bs, D), lambda e, i: (e, i, 0)),
            pl.BlockSpec((1, D, D), lambda e, i: (e, 0, 0)),
            pl.BlockSpec((1, 1, D), lambda e, i: (e, 0, 0)),
        ],
        out_specs=pl.BlockSpec((1, bs, D), lambda e, i: (e, i, 0)),
        out_shape=jax.ShapeDtypeStruct((2, n, D), jnp.float32),
    )(X, EMB, WL, BL)


def _segment_sums(x_flat, src_all, dst_all, npad, e):
    per_tile = e // NS
    ch_sz = 64
    nfull = per_tile // ch_sz - per_tile // ch_sz % 4
    tail = per_tile - nfull * ch_sz
    rows_out = npad // NS

    @functools.partial(
        pl.kernel,
        out_type=jax.ShapeDtypeStruct((2 * npad, D), jnp.float32),
        mesh=_MESH(),
        scratch_types=[pltpu.VMEM((4, ch_sz), jnp.int32),
                       pltpu.VMEM((4, ch_sz), jnp.int32),
                       pltpu.VMEM((4, ch_sz, D), jnp.float32),
                       pltpu.VMEM((tail,), jnp.int32),
                       pltpu.VMEM((tail,), jnp.int32),
                       pltpu.VMEM((tail, D), jnp.float32),
                       pltpu.VMEM_SHARED((npad, D), jnp.float32),
                       pltpu.SemaphoreType.DMA,
                       pltpu.SemaphoreType.DMA,
                       pltpu.SemaphoreType.DMA,
                       pltpu.SemaphoreType.DMA,
                       pltpu.SemaphoreType.DMA,
                       pltpu.SemaphoreType.DMA,
                       pltpu.SemaphoreType.DMA,
                       pltpu.SemaphoreType.DMA,
                       pltpu.SemaphoreType.DMA,
                       pltpu.SemaphoreType.DMA,
                       pltpu.SemaphoreType.DMA,
                       pltpu.SemaphoreType.DMA],
    )
    def k(x_h, src_h, dst_h, out_h, idx_s4, idx_d4, rows4,
          idx_st, idx_dt, rows_t, acc,
          sg0, sg1, sg2, sg3, si0, si1, si2, si3, sc0, sc1, sc2, sc3):
        c = lax.axis_index("c")
        s = lax.axis_index("s")
        sem_g = (sg0, sg1, sg2, sg3)
        sem_i = (si0, si1, si2, si3)
        sem_s = (sc0, sc1, sc2, sc3)

        def zbody(r, carry):
            for j in range(D // 16):
                rows4[0, r, pl.ds(j * 16, 16)] = jnp.zeros((16,), jnp.float32)
            return carry
        lax.fori_loop(0, ch_sz, zbody, 0)
        for t in range(rows_out // ch_sz):
            pltpu.sync_copy(rows4.at[0],
                            acc.at[pl.ds(s * rows_out + t * ch_sz, ch_sz)])
        plsc.subcore_barrier()

        base = c * e + s * per_tile

        def idx_load(chk, b):
            off = base + chk * ch_sz
            pltpu.async_copy(src_h.at[pl.ds(off, ch_sz)], idx_s4.at[b],
                             sem_i[b])
            pltpu.async_copy(dst_h.at[pl.ds(off, ch_sz)], idx_d4.at[b],
                             sem_i[b])

        def idx_wait(chk, b):
            off = base + chk * ch_sz
            pltpu.make_async_copy(src_h.at[pl.ds(off, ch_sz)], idx_s4.at[b],
                                  sem_i[b]).wait()
            pltpu.make_async_copy(dst_h.at[pl.ds(off, ch_sz)], idx_d4.at[b],
                                  sem_i[b]).wait()

        def gather_start(b):
            pltpu.async_copy(x_h.at[idx_s4.at[b]], rows4.at[b], sem_g[b])

        def gather_wait(b):
            pltpu.make_async_copy(x_h.at[idx_s4.at[b]], rows4.at[b],
                                  sem_g[b]).wait()

        def scatter_start(b):
            pltpu.async_copy(rows4.at[b], acc.at[idx_d4.at[b]], sem_s[b],
                             add=True)

        def scatter_wait(b):
            pltpu.make_async_copy(rows4.at[b], acc.at[idx_d4.at[b]],
                                  sem_s[b]).wait()

        idx_load(0, 0)
        idx_wait(0, 0)
        gather_start(0)
        idx_load(1, 1)

        def quad(i4, carry):
            for j in range(4):
                chk = i4 * 4 + j

                @pl.when(chk >= 2)
                def _():
                    scatter_wait((j + 2) % 4)

                @pl.when(chk + 2 < nfull)
                def _():
                    idx_load(chk + 2, (j + 2) % 4)

                gather_wait(j)
                scatter_start(j)

                @pl.when(chk + 1 < nfull)
                def _():
                    idx_wait(chk + 1, (j + 1) % 4)
                    gather_start((j + 1) % 4)
            return carry
        lax.fori_loop(0, nfull // 4, quad, 0)
        scatter_wait((nfull - 2) % 4)
        scatter_wait((nfull - 1) % 4)

        if tail:
            offt = base + nfull * ch_sz
            pltpu.sync_copy(src_h.at[pl.ds(offt, tail)], idx_st)
            pltpu.sync_copy(dst_h.at[pl.ds(offt, tail)], idx_dt)
            pltpu.async_copy(x_h.at[idx_st], rows_t, sg0).wait()
            pltpu.sync_copy(rows_t, acc.at[idx_dt], add=True)

        plsc.subcore_barrier()
        pltpu.sync_copy(acc.at[pl.ds(s * rows_out, rows_out)],
                        out_h.at[pl.ds(c * npad + s * rows_out, rows_out)])

    return k(x_flat, src_all, dst_all)


def _conv_tc(X1, MSG, WS, WM, BB):
    n = X1.shape[1]
    bs = 2000
    grid = (2, n // bs)

    def body(x_ref, m_ref, ws_ref, wm_ref, b_ref, o_ref):
        acc = (jnp.dot(x_ref[0], ws_ref[0], preferred_element_type=jnp.float32)
               + jnp.dot(m_ref[0], wm_ref[0], preferred_element_type=jnp.float32)
               + b_ref[0])
        o_ref[...] = jnp.maximum(acc, 0.0)[None]

    return pl.pallas_call(
        body,
        grid=grid,
        in_specs=[
            pl.BlockSpec((1, bs, D), lambda e, i: (e, i, 0)),
            pl.BlockSpec((1, bs, D), lambda e, i: (e, i, 0)),
            pl.BlockSpec((1, D, D), lambda e, i: (e, 0, 0)),
            pl.BlockSpec((1, D, D), lambda e, i: (e, 0, 0)),
            pl.BlockSpec((1, 1, D), lambda e, i: (e, 0, 0)),
        ],
        out_specs=pl.BlockSpec((1, bs, D), lambda e, i: (e, i, 0)),
        out_shape=jax.ShapeDtypeStruct((2, n, D), jnp.float32),
    )(X1, MSG, WS, WM, BB)


def _classifier(h_flat, i0, i1_off):
    l = i0.shape[0]
    per_tile = l // NW
    chunk = 128
    nchunks = per_tile // chunk
    groups = chunk // 16

    @functools.partial(
        pl.kernel,
        out_type=jax.ShapeDtypeStruct((l,), jnp.float32),
        mesh=_MESH(),
        scratch_types=[pltpu.VMEM((chunk,), jnp.int32),
                       pltpu.VMEM((chunk,), jnp.int32),
                       pltpu.VMEM((chunk, D), jnp.float32),
                       pltpu.VMEM((chunk, D), jnp.float32),
                       pltpu.VMEM((256,), jnp.float32),
                       pltpu.VMEM((per_tile,), jnp.float32),
                       pltpu.SemaphoreType.DMA],
        compiler_params=pltpu.CompilerParams(needs_layout_passes=False),
    )
    def k(h_h, i0_h, i1_h, pred_h,
          idx0, idx1, urows, arows, tbuf, out_v, sem):
        c = lax.axis_index("c")
        s = lax.axis_index("s")
        wid = s * NC + c
        base = wid * per_tile
        lane = lax.iota(jnp.int32, 16)

        def body(i, carry):
            off = base + i * chunk
            pltpu.sync_copy(i0_h.at[pl.ds(off, chunk)], idx0)
            pltpu.sync_copy(i1_h.at[pl.ds(off, chunk)], idx1)
            pltpu.async_copy(h_h.at[idx0], urows, sem).wait()
            pltpu.async_copy(h_h.at[idx1], arows, sem).wait()

            def group(g, carry2):
                def edge(ei, carry3):
                    r = g * 16 + ei
                    acc = urows[r, pl.ds(0, 16)] * arows[r, pl.ds(0, 16)]
                    for j in range(1, 8):
                        acc = acc + (urows[r, pl.ds(j * 16, 16)]
                                     * arows[r, pl.ds(j * 16, 16)])
                    tbuf[pl.ds(ei * 16, 16)] = acc
                    return carry3
                lax.fori_loop(0, 16, edge, 0)
                row16 = lane * 16
                r16 = plsc.load_gather(tbuf, [row16])
                for col in range(1, 16):
                    r16 = r16 + plsc.load_gather(tbuf, [row16 + col])
                out_v[pl.ds(i * chunk + g * 16, 16)] = r16
                return carry2
            lax.fori_loop(0, groups, group, 0)
            return carry
        lax.fori_loop(0, nchunks, body, 0)
        pltpu.sync_copy(out_v, pred_h.at[pl.ds(base, per_tile)])

    return k(h_flat, i0, i1_off)


def kernel(user_n_id, user_x, app_n_id, app_x, edge_index_u2a, edge_index_a2u,
           edge_label_index, user_emb, app_emb, user_lin_w, user_lin_b,
           app_lin_w, app_lin_b, w_self_user, w_msg_user, b_user,
           w_self_app, w_msg_app, b_app):
    n = user_emb.shape[0]
    e = edge_index_u2a.shape[1]
    npad = -(-n // (NS * 80)) * (NS * 80)

    tables = jnp.concatenate([user_emb, app_emb])
    uid = jnp.pad(user_n_id.astype(jnp.int32), (0, npad - n))
    aid = jnp.pad(app_n_id.astype(jnp.int32), (0, npad - n)) + n
    ids_flat = jnp.concatenate([uid, aid])

    emb_pad = lax.optimization_barrier(
        _emb_gather(tables, ids_flat, npad))

    X = jnp.stack([user_x, app_x])
    EMB = emb_pad.reshape(2, npad, D)[:, :n]
    WL = jnp.stack([user_lin_w, app_lin_w])
    BL = jnp.stack([user_lin_b, app_lin_b])[:, None, :]
    X1 = lax.optimization_barrier(_lin_tc(X, EMB, WL, BL))

    su = edge_index_u2a[0].astype(jnp.int32)
    du = edge_index_u2a[1].astype(jnp.int32)
    sa = edge_index_a2u[0].astype(jnp.int32) + n
    da = edge_index_a2u[1].astype(jnp.int32)
    src_all = jnp.concatenate([su, sa])
    dst_all = jnp.concatenate([du, da])
    msgs = lax.optimization_barrier(
        _segment_sums(X1.reshape(2 * n, D), src_all, dst_all,
                      npad, e))
    msgs = msgs.reshape(2, npad, D)[:, :n]

    MSG = jnp.stack([msgs[1], msgs[0]])
    WS = jnp.stack([w_self_user, w_self_app])
    WM = jnp.stack([w_msg_user, w_msg_app])
    BB = jnp.stack([b_user, b_app])[:, None, :]
    H = lax.optimization_barrier(_conv_tc(X1, MSG, WS, WM, BB))

    i0 = edge_label_index[0].astype(jnp.int32)
    i1 = edge_label_index[1].astype(jnp.int32) + n
    return _classifier(H.reshape(2 * n, D), i0, i1)


# speedup vs baseline: 1.3932x; 1.3932x over previous
import functools

import jax
import jax.numpy as jnp
from jax import lax
from jax.experimental import pallas as pl
from jax.experimental.pallas import tpu as pltpu
from jax.experimental.pallas import tpu_sc as plsc

NC, NS = 2, 16
NW = NC * NS
D = 128

_MESH = functools.partial(
    plsc.VectorSubcoreMesh, core_axis_name="c", subcore_axis_name="s",
    num_cores=NC, num_subcores=NS)


def _emb_gather(tables, ids_flat, npad):
    rows_per_tile = npad // NS
    chunk = 128
    nchunks = rows_per_tile // chunk

    @functools.partial(
        pl.kernel,
        out_type=jax.ShapeDtypeStruct((2 * npad, D), jnp.float32),
        mesh=_MESH(),
        scratch_types=[pltpu.VMEM((rows_per_tile,), jnp.int32),
                       pltpu.VMEM((nchunks, chunk, D), jnp.float32),
                       pltpu.SemaphoreType.DMA,
                       pltpu.SemaphoreType.DMA],
    )
    def k(tab, ids, out, idx_v, rows_v, sem_g, sem_w):
        c = lax.axis_index("c")
        s = lax.axis_index("s")
        base = c * npad + s * rows_per_tile

        pltpu.sync_copy(ids.at[pl.ds(base, rows_per_tile)], idx_v)
        for j in range(nchunks):
            pltpu.async_copy(tab.at[idx_v.at[pl.ds(j * chunk, chunk)]],
                             rows_v.at[j], sem_g)
        for j in range(nchunks):
            pltpu.make_async_copy(tab.at[idx_v.at[pl.ds(j * chunk, chunk)]],
                                  rows_v.at[j], sem_g).wait()
        for j in range(nchunks):
            pltpu.async_copy(rows_v.at[j],
                             out.at[pl.ds(base + j * chunk, chunk)], sem_w)
        for j in range(nchunks):
            pltpu.make_async_copy(rows_v.at[j],
                                  out.at[pl.ds(base + j * chunk, chunk)],
                                  sem_w).wait()

    return k(tables, ids_flat)


def _lin_tc(X, EMB, WL, BL):
    n = X.shape[1]
    bs = 2000
    grid = (2, n // bs)

    def body(x_ref, e_ref, w_ref, b_ref, o_ref):
        o_ref[...] = (jnp.dot(x_ref[0], w_ref[0],
                              preferred_element_type=jnp.float32)
                      + b_ref[0] + e_ref[0])[None]

    return pl.pallas_call(
        body,
        grid=grid,
        in_specs=[
            pl.BlockSpec((1, bs, D), lambda e, i: (e, i, 0)),
            pl.BlockSpec((1, bs, D), lambda e, i: (e, i, 0)),
            pl.BlockSpec((1, D, D), lambda e, i: (e, 0, 0)),
            pl.BlockSpec((1, 1, D), lambda e, i: (e, 0, 0)),
        ],
        out_specs=pl.BlockSpec((1, bs, D), lambda e, i: (e, i, 0)),
        out_shape=jax.ShapeDtypeStruct((2, n, D), jnp.float32),
    )(X, EMB, WL, BL)


def _segment_sums(x_flat, src_all, dst_all, npad, e):
    per_tile = e // NS
    ch_sz = 128
    nfull = per_tile // ch_sz - per_tile // ch_sz % 2
    tail = per_tile - nfull * ch_sz
    rows_out = npad // NS

    @functools.partial(
        pl.kernel,
        out_type=jax.ShapeDtypeStruct((2 * npad, D), jnp.float32),
        mesh=_MESH(),
        scratch_types=[pltpu.VMEM((2, ch_sz), jnp.int32),
                       pltpu.VMEM((2, ch_sz), jnp.int32),
                       pltpu.VMEM((2, ch_sz, D), jnp.float32),
                       pltpu.VMEM((tail,), jnp.int32),
                       pltpu.VMEM((tail,), jnp.int32),
                       pltpu.VMEM((tail, D), jnp.float32),
                       pltpu.VMEM_SHARED((npad, D), jnp.float32),
                       pltpu.SemaphoreType.DMA,
                       pltpu.SemaphoreType.DMA,
                       pltpu.SemaphoreType.DMA,
                       pltpu.SemaphoreType.DMA,
                       pltpu.SemaphoreType.DMA,
                       pltpu.SemaphoreType.DMA,
                       pltpu.SemaphoreType.DMA],
    )
    def k(x_h, src_h, dst_h, out_h, idx_s2, idx_d2, rows2,
          idx_st, idx_dt, rows_t, acc,
          sg0, sg1, si0, si1, sd0, sd1, sd2):
        c = lax.axis_index("c")
        s = lax.axis_index("s")
        sem_g = (sg0, sg1)
        sem_i = (si0, si1)
        sem_d = (sd0, sd1)

        def zbody(r, carry):
            for j in range(D // 16):
                rows2[0, r, pl.ds(j * 16, 16)] = jnp.zeros((16,), jnp.float32)
            return carry
        lax.fori_loop(0, ch_sz, zbody, 0)
        for t in range(rows_out // ch_sz):
            pltpu.sync_copy(rows2.at[0],
                            acc.at[pl.ds(s * rows_out + t * ch_sz, ch_sz)])
        plsc.subcore_barrier()

        base = c * e + s * per_tile

        def idx_load(chk, b):
            off = base + chk * ch_sz
            pltpu.async_copy(src_h.at[pl.ds(off, ch_sz)], idx_s2.at[b],
                             sem_i[b])
            pltpu.async_copy(dst_h.at[pl.ds(off, ch_sz)], idx_d2.at[b],
                             sem_d[b])

        def idx_wait(chk, b):
            off = base + chk * ch_sz
            pltpu.make_async_copy(src_h.at[pl.ds(off, ch_sz)], idx_s2.at[b],
                                  sem_i[b]).wait()
            pltpu.make_async_copy(dst_h.at[pl.ds(off, ch_sz)], idx_d2.at[b],
                                  sem_d[b]).wait()

        def gather_start(b):
            pltpu.async_copy(x_h.at[idx_s2.at[b]], rows2.at[b], sem_g[b])

        def gather_wait(b):
            pltpu.make_async_copy(x_h.at[idx_s2.at[b]], rows2.at[b],
                                  sem_g[b]).wait()

        idx_load(0, 0)
        idx_wait(0, 0)
        gather_start(0)
        idx_load(1, 1)

        def pair(i2, carry):
            for b in (0, 1):
                chk = i2 * 2 + b
                nb = 1 - b

                @pl.when(chk + 1 < nfull)
                def _():
                    idx_wait(chk + 1, nb)
                    gather_start(nb)

                gather_wait(b)
                pltpu.sync_copy(rows2.at[b], acc.at[idx_d2.at[b]], add=True)

                @pl.when(chk + 2 < nfull)
                def _():
                    idx_load(chk + 2, b)
            return carry
        lax.fori_loop(0, nfull // 2, pair, 0)

        if tail:
            offt = base + nfull * ch_sz
            pltpu.sync_copy(src_h.at[pl.ds(offt, tail)], idx_st)
            pltpu.sync_copy(dst_h.at[pl.ds(offt, tail)], idx_dt)
            pltpu.async_copy(x_h.at[idx_st], rows_t, sg0).wait()
            pltpu.sync_copy(rows_t, acc.at[idx_dt], add=True)

        plsc.subcore_barrier()
        pltpu.sync_copy(acc.at[pl.ds(s * rows_out, rows_out)],
                        out_h.at[pl.ds(c * npad + s * rows_out, rows_out)])

    return k(x_flat, src_all, dst_all)


def _conv_tc(X1, MSG, WS, WM, BB):
    n = X1.shape[1]
    bs = 2000
    grid = (2, n // bs)

    def body(x_ref, m_ref, ws_ref, wm_ref, b_ref, o_ref):
        acc = (jnp.dot(x_ref[0], ws_ref[0], preferred_element_type=jnp.float32)
               + jnp.dot(m_ref[0], wm_ref[0], preferred_element_type=jnp.float32)
               + b_ref[0])
        o_ref[...] = jnp.maximum(acc, 0.0)[None]

    return pl.pallas_call(
        body,
        grid=grid,
        in_specs=[
            pl.BlockSpec((1, bs, D), lambda e, i: (e, i, 0)),
            pl.BlockSpec((1, bs, D), lambda e, i: (e, i, 0)),
            pl.BlockSpec((1, D, D), lambda e, i: (e, 0, 0)),
            pl.BlockSpec((1, D, D), lambda e, i: (e, 0, 0)),
            pl.BlockSpec((1, 1, D), lambda e, i: (e, 0, 0)),
        ],
        out_specs=pl.BlockSpec((1, bs, D), lambda e, i: (e, i, 0)),
        out_shape=jax.ShapeDtypeStruct((2, n, D), jnp.float32),
    )(X1, MSG, WS, WM, BB)


def _classifier(h_flat, i0, i1_off):
    l = i0.shape[0]
    per_tile = l // NW
    chunk = 128
    nchunks = per_tile // chunk
    groups = chunk // 16

    @functools.partial(
        pl.kernel,
        out_type=jax.ShapeDtypeStruct((l,), jnp.float32),
        mesh=_MESH(),
        scratch_types=[pltpu.VMEM((2, chunk), jnp.int32),
                       pltpu.VMEM((2, chunk), jnp.int32),
                       pltpu.VMEM((2, chunk, D), jnp.float32),
                       pltpu.VMEM((2, chunk, D), jnp.float32),
                       pltpu.VMEM((256,), jnp.float32),
                       pltpu.VMEM((per_tile,), jnp.float32),
                       pltpu.SemaphoreType.DMA,
                       pltpu.SemaphoreType.DMA,
                       pltpu.SemaphoreType.DMA,
                       pltpu.SemaphoreType.DMA],
        compiler_params=pltpu.CompilerParams(needs_layout_passes=False),
    )
    def k(h_h, i0_h, i1_h, pred_h,
          idx0, idx1, urows, arows, tbuf, out_v, sg0, sg1, si0, si1):
        c = lax.axis_index("c")
        s = lax.axis_index("s")
        sem_g = (sg0, sg1)
        sem_i = (si0, si1)
        wid = s * NC + c
        base = wid * per_tile
        lane = lax.iota(jnp.int32, 16)

        def idx_load(chk, b):
            off = base + chk * chunk
            pltpu.async_copy(i0_h.at[pl.ds(off, chunk)], idx0.at[b], sem_i[b])
            pltpu.async_copy(i1_h.at[pl.ds(off, chunk)], idx1.at[b], sem_i[b])

        def idx_wait(chk, b):
            off = base + chk * chunk
            pltpu.make_async_copy(i0_h.at[pl.ds(off, chunk)], idx0.at[b],
                                  sem_i[b]).wait()
            pltpu.make_async_copy(i1_h.at[pl.ds(off, chunk)], idx1.at[b],
                                  sem_i[b]).wait()

        def gather_start(b):
            pltpu.async_copy(h_h.at[idx0.at[b]], urows.at[b], sem_g[b])
            pltpu.async_copy(h_h.at[idx1.at[b]], arows.at[b], sem_g[b])

        def gather_wait(b):
            pltpu.make_async_copy(h_h.at[idx0.at[b]], urows.at[b],
                                  sem_g[b]).wait()
            pltpu.make_async_copy(h_h.at[idx1.at[b]], arows.at[b],
                                  sem_g[b]).wait()

        idx_load(0, 0)
        idx_wait(0, 0)
        gather_start(0)
        idx_load(1, 1)

        def pair(i2, carry):
            for b in (0, 1):
                chk = i2 * 2 + b
                nb = 1 - b

                @pl.when(chk + 1 < nchunks)
                def _():
                    idx_wait(chk + 1, nb)
                    gather_start(nb)

                gather_wait(b)

                def group(g, carry2):
                    r0 = g * 16
                    for ei in range(16):
                        acc = (urows[b, r0 + ei, pl.ds(0, 16)]
                               * arows[b, r0 + ei, pl.ds(0, 16)])
                        for j in range(1, 8):
                            acc = acc + (urows[b, r0 + ei, pl.ds(j * 16, 16)]
                                         * arows[b, r0 + ei, pl.ds(j * 16, 16)])
                        tbuf[pl.ds(ei * 16, 16)] = acc
                    row16 = lane * 16
                    r16 = plsc.load_gather(tbuf, [row16])
                    for col in range(1, 16):
                        r16 = r16 + plsc.load_gather(tbuf, [row16 + col])
                    out_v[pl.ds(chk * chunk + g * 16, 16)] = r16
                    return carry2
                lax.fori_loop(0, groups, group, 0)

                @pl.when(chk + 2 < nchunks)
                def _():
                    idx_load(chk + 2, b)
            return carry
        lax.fori_loop(0, nchunks // 2, pair, 0)
        pltpu.sync_copy(out_v, pred_h.at[pl.ds(base, per_tile)])

    return k(h_flat, i0, i1_off)


def kernel(user_n_id, user_x, app_n_id, app_x, edge_index_u2a, edge_index_a2u,
           edge_label_index, user_emb, app_emb, user_lin_w, user_lin_b,
           app_lin_w, app_lin_b, w_self_user, w_msg_user, b_user,
           w_self_app, w_msg_app, b_app):
    n = user_emb.shape[0]
    e = edge_index_u2a.shape[1]
    npad = -(-n // (NS * 80)) * (NS * 80)

    tables = jnp.concatenate([user_emb, app_emb])
    uid = jnp.pad(user_n_id.astype(jnp.int32), (0, npad - n))
    aid = jnp.pad(app_n_id.astype(jnp.int32), (0, npad - n)) + n
    ids_flat = jnp.concatenate([uid, aid])

    emb_pad = lax.optimization_barrier(
        _emb_gather(tables, ids_flat, npad))

    X = jnp.stack([user_x, app_x])
    EMB = emb_pad.reshape(2, npad, D)[:, :n]
    WL = jnp.stack([user_lin_w, app_lin_w])
    BL = jnp.stack([user_lin_b, app_lin_b])[:, None, :]
    X1 = lax.optimization_barrier(_lin_tc(X, EMB, WL, BL))

    su = edge_index_u2a[0].astype(jnp.int32)
    du = edge_index_u2a[1].astype(jnp.int32)
    sa = edge_index_a2u[0].astype(jnp.int32) + n
    da = edge_index_a2u[1].astype(jnp.int32)
    src_all = jnp.concatenate([su, sa])
    dst_all = jnp.concatenate([du, da])
    msgs = lax.optimization_barrier(
        _segment_sums(X1.reshape(2 * n, D), src_all, dst_all,
                      npad, e))
    msgs = msgs.reshape(2, npad, D)[:, :n]

    MSG = jnp.stack([msgs[1], msgs[0]])
    WS = jnp.stack([w_self_user, w_self_app])
    WM = jnp.stack([w_msg_user, w_msg_app])
    BB = jnp.stack([b_user, b_app])[:, None, :]
    H = lax.optimization_barrier(_conv_tc(X1, MSG, WS, WM, BB))

    i0 = edge_label_index[0].astype(jnp.int32)
    i1 = edge_label_index[1].astype(jnp.int32) + n
    return _classifier(H.reshape(2 * n, D), i0, i1)
